# baseline (speedup 1.0000x reference)
import dataclasses
import functools

import jax
import jax.numpy as jnp
from jax import lax
from jax.experimental import pallas as pl
from jax.experimental.pallas import tpu as pltpu
from jax.experimental.pallas import tpu_sc as plsc

_H = 256
_NG = 10000
_NGP = 10240
_EG = 320000
_MG = 512
_AF = 128
_BF = 144
_DEPTH = 3

_N_SUBCORES = 16
_N_CORES = 2


def _sc_params():
    cp = pltpu.CompilerParams()
    if "needs_layout_passes" in pltpu.CompilerParams.__dataclass_fields__:
        cp = dataclasses.replace(cp, needs_layout_passes=False)
    return cp


def _bdot(x, w):
    return jnp.dot(x.astype(jnp.bfloat16), w.astype(jnp.bfloat16),
                   preferred_element_type=jnp.float32)


def _mm_body(x_ref, w_ref, o_ref, *, relu_in):
    x = x_ref[...]
    if relu_in:
        x = jnp.maximum(x, 0.0)
    o_ref[...] = _bdot(x, w_ref[...])


def _mm(x, w, relu_in=False, block=2000):
    e, k = x.shape
    n = w.shape[1]
    return pl.pallas_call(
        functools.partial(_mm_body, relu_in=relu_in),
        grid=(e // block,),
        in_specs=[pl.BlockSpec((block, k), lambda i: (i, 0)),
                  pl.BlockSpec((k, n), lambda i: (0, 0))],
        out_specs=pl.BlockSpec((block, n), lambda i: (i, 0)),
        out_shape=jax.ShapeDtypeStruct((e, n), jnp.float32),
    )(x, w)


def _mm_h2_body(x_ref, d_ref, w_ref, o_ref, o2_ref, *, relu_in):
    if d_ref is None:
        x = jnp.maximum(x_ref[...], 0.0) if relu_in else x_ref[...]
    else:
        x = jnp.maximum(x_ref[...] + d_ref[...], 0.0)
    hm = _bdot(x, w_ref[...])
    o_ref[...] = hm
    o2_ref[...] = hm.reshape(o2_ref.shape)


def _mm_h2(inp, w, d=None, relu_in=True, block=2000):
    e, k = inp.shape
    n = w.shape[1]
    in_specs = [pl.BlockSpec((block, k), lambda i: (i, 0))]
    args = [inp]
    if d is not None:
        in_specs.append(pl.BlockSpec((block, k), lambda i: (i, 0)))
        args.append(d)
    in_specs.append(pl.BlockSpec((k, n), lambda i: (0, 0)))
    args.append(w)
    body = ((lambda x, dd, ww, o, o2: _mm_h2_body(x, dd, ww, o, o2,
                                                  relu_in=relu_in))
            if d is not None else
            (lambda x, ww, o, o2: _mm_h2_body(x, None, ww, o, o2,
                                              relu_in=relu_in)))
    return pl.pallas_call(
        body,
        grid=(e // block,),
        in_specs=in_specs,
        out_specs=[pl.BlockSpec((block, n), lambda i: (i, 0)),
                   pl.BlockSpec((2 * block, n // 2), lambda i: (i, 0))],
        out_shape=[jax.ShapeDtypeStruct((e, n), jnp.float32),
                   jax.ShapeDtypeStruct((2 * e, n // 2), jnp.float32)],
    )(*args)


def _mm_i2_body(a_ref, b_ref, w_ref, o_ref, *, nblk):
    sel = pl.program_id(0) < nblk
    x = jnp.where(sel, a_ref[...], b_ref[...])
    o_ref[...] = _bdot(x, w_ref[...])


def _mm_i2(xa, xb, w, block=2000):
    e, k = xa.shape
    n = w.shape[1]
    nblk = e // block
    return pl.pallas_call(
        functools.partial(_mm_i2_body, nblk=nblk),
        grid=(2 * nblk,),
        in_specs=[
            pl.BlockSpec((block, k),
                         lambda i: (jnp.where(i < nblk, i, 0), 0)),
            pl.BlockSpec((block, k),
                         lambda i: (jnp.where(i >= nblk, i - nblk, 0), 0)),
            pl.BlockSpec((k, n), lambda i: (0, 0)),
        ],
        out_specs=pl.BlockSpec((block, n), lambda i: (i, 0)),
        out_shape=jax.ShapeDtypeStruct((2 * e, n), jnp.float32),
    )(xa, xb, w)


def _mm_hd_body(inp_ref, d_ref, w_ref, o_ref):
    x = jnp.maximum(inp_ref[...] + d_ref[...], 0.0)
    o_ref[...] = _bdot(x, w_ref[...])


def _mm_hd(inp, d, w, block=2000):
    e, k = inp.shape
    n = w.shape[1]
    return pl.pallas_call(
        _mm_hd_body,
        grid=(e // block,),
        in_specs=[pl.BlockSpec((block, k), lambda i: (i, 0)),
                  pl.BlockSpec((block, k), lambda i: (i, 0)),
                  pl.BlockSpec((k, n), lambda i: (0, 0))],
        out_specs=pl.BlockSpec((block, n), lambda i: (i, 0)),
        out_shape=jax.ShapeDtypeStruct((e, n), jnp.float32),
    )(inp, d, w)


def _msg_body(inp_ref, d_ref, o_ref):
    o_ref[...] = jnp.maximum(inp_ref[...] + d_ref[...], 0.0)


def _msg(inp, d, block=2000):
    e, k = inp.shape
    return pl.pallas_call(
        _msg_body,
        grid=(e // block,),
        in_specs=[pl.BlockSpec((block, k), lambda i: (i, 0)),
                  pl.BlockSpec((block, k), lambda i: (i, 0))],
        out_specs=pl.BlockSpec((block, k), lambda i: (i, 0)),
        out_shape=jax.ShapeDtypeStruct((e, k), jnp.float32),
    )(inp, d)


def _zero_stripe(buf, accum, s, npt, zrows):
    @pl.loop(0, zrows)
    def _(i):
        for j in range(8):
            buf[i, pl.ds(j * 16, 16)] = jnp.zeros((16,), jnp.float32)

    for z in range(npt // zrows):
        pltpu.sync_copy(buf.at[pl.ds(0, zrows)],
                        accum.at[pl.ds(s * npt + z * zrows, zrows)])


def _segsum_sc(hm, dst_local, num_graphs):
    e_tot = hm.shape[0]
    epg = e_tot // num_graphs
    ept = epg // _N_SUBCORES
    chunk = 160
    nchunk = ept // chunk
    npt = _NGP // _N_SUBCORES
    mesh = plsc.VectorSubcoreMesh(core_axis_name="c", subcore_axis_name="s")

    @functools.partial(
        pl.kernel,
        out_type=jax.ShapeDtypeStruct((num_graphs * _NGP, _H), jnp.float32),
        mesh=mesh,
        scratch_types=[
            pltpu.VMEM_SHARED((_NGP, 128), jnp.float32),
            pltpu.VMEM((chunk, 128), jnp.float32),
            pltpu.VMEM((chunk, 128), jnp.float32),
            pltpu.VMEM((chunk,), jnp.int32),
            pltpu.VMEM((chunk,), jnp.int32),
            pltpu.SemaphoreType.DMA,
            pltpu.SemaphoreType.DMA,
        ],
    )
    def k(hm_hbm, dst_hbm, out_hbm, accum,
          rows_a, rows_b, idx_a, idx_b, sem_a, sem_b):
        c = lax.axis_index("c")
        s = lax.axis_index("s")
        bufs = ((rows_a, idx_a, sem_a), (rows_b, idx_b, sem_b))

        for g in range(num_graphs):
            _zero_stripe(rows_a, accum, s, npt, chunk)
            plsc.subcore_barrier()

            def issue(kk, b):
                rows, idx, sem = bufs[b]
                e0 = g * epg + s * ept + kk * chunk
                pltpu.async_copy(dst_hbm.at[pl.ds(e0, chunk)], idx, sem)
                pltpu.async_copy(
                    hm_hbm.at[pl.ds(e0, chunk), pl.ds(c * 128, 128)],
                    rows, sem)

            def drain(b):
                rows, idx, sem = bufs[b]
                pltpu.make_async_copy(dst_hbm.at[pl.ds(0, chunk)], idx,
                                      sem).wait()
                pltpu.make_async_copy(
                    hm_hbm.at[pl.ds(0, chunk), pl.ds(0, 128)], rows,
                    sem).wait()
                pltpu.sync_copy(rows, accum.at[idx], add=True)

            issue(0, 0)
            issue(1, 1)

            @pl.loop(0, (nchunk + 1) // 2)
            def _(q):
                kk = 2 * q
                drain(0)

                @pl.when(kk + 2 < nchunk)
                def _():
                    issue(kk + 2, 0)

                @pl.when(kk + 1 < nchunk)
                def _():
                    drain(1)

                @pl.when(kk + 3 < nchunk)
                def _():
                    issue(kk + 3, 1)

            plsc.subcore_barrier()
            pltpu.sync_copy(
                accum.at[pl.ds(s * npt, npt)],
                out_hbm.at[pl.ds(g * _NGP + s * npt, npt), pl.ds(c * 128, 128)])
            plsc.subcore_barrier()

    return k(hm, dst_local)


def _segdiff_sc(hm, hm2, dst_local, src_local, rev_g, num_graphs):
    e_tot = hm.shape[0]
    epg = e_tot // num_graphs
    ept = epg // _N_SUBCORES
    chunk = 80
    nchunk = ept // chunk
    npt = _NGP // _N_SUBCORES
    ng_all = num_graphs * _NGP
    mesh = plsc.VectorSubcoreMesh(core_axis_name="c", subcore_axis_name="s")

    @functools.partial(
        pl.kernel,
        out_type=[jax.ShapeDtypeStruct((e_tot, _H), jnp.float32),
                  jax.ShapeDtypeStruct((2 * ng_all, 128), jnp.float32)],
        mesh=mesh,
        scratch_types=[
            pltpu.VMEM_SHARED((_NGP, 128), jnp.float32),
            pltpu.VMEM((chunk, 128), jnp.float32),
            pltpu.VMEM((chunk, 128), jnp.float32),
            pltpu.VMEM((chunk, 128), jnp.float32),
            pltpu.VMEM((chunk, 128), jnp.float32),
            pltpu.VMEM((chunk,), jnp.int32),
            pltpu.VMEM((chunk,), jnp.int32),
            pltpu.VMEM((chunk,), jnp.int32),
            pltpu.VMEM((chunk,), jnp.int32),
            pltpu.SemaphoreType.DMA,
            pltpu.SemaphoreType.DMA,
            pltpu.SemaphoreType.DMA,
            pltpu.SemaphoreType.DMA,
        ],
    )
    def k(hm_hbm, dst_hbm, src_hbm, rev_hbm, hm2_hbm, d_hbm, ah2_hbm, accum,
          b1, b2, b3, b4, ia_s, ia_r, ib_s, ib_r, sa, sb, swa, swb):
        c = lax.axis_index("c")
        s = lax.axis_index("s")
        p1bufs = ((b1, ia_s, sa), (b3, ib_s, sb))
        p2bufs = ((b1, b2, ia_s, ia_r, sa, swa), (b3, b4, ib_s, ib_r, sb, swb))

        for g in range(num_graphs):
            _zero_stripe(b1, accum, s, npt, chunk)
            plsc.subcore_barrier()

            def issue1(kk, b):
                rows, idx, sem = p1bufs[b]
                e0 = g * epg + s * ept + kk * chunk
                pltpu.async_copy(dst_hbm.at[pl.ds(e0, chunk)], idx, sem)
                pltpu.async_copy(
                    hm_hbm.at[pl.ds(e0, chunk), pl.ds(c * 128, 128)],
                    rows, sem)

            def drain1(b):
                rows, idx, sem = p1bufs[b]
                pltpu.make_async_copy(dst_hbm.at[pl.ds(0, chunk)], idx,
                                      sem).wait()
                pltpu.make_async_copy(
                    hm_hbm.at[pl.ds(0, chunk), pl.ds(0, 128)], rows,
                    sem).wait()
                pltpu.sync_copy(rows, accum.at[idx], add=True)

            issue1(0, 0)
            issue1(1, 1)

            @pl.loop(0, (nchunk + 1) // 2)
            def _(q):
                kk = 2 * q
                drain1(0)

                @pl.when(kk + 2 < nchunk)
                def _():
                    issue1(kk + 2, 0)

                @pl.when(kk + 1 < nchunk)
                def _():
                    drain1(1)

                @pl.when(kk + 3 < nchunk)
                def _():
                    issue1(kk + 3, 1)

            plsc.subcore_barrier()
            pltpu.sync_copy(
                accum.at[pl.ds(s * npt, npt)],
                ah2_hbm.at[pl.ds(c * ng_all + g * _NGP + s * npt, npt)])
            plsc.subcore_barrier()

            e00 = g * epg + s * ept
            pltpu.async_copy(
                b1, d_hbm.at[pl.ds(e00, chunk), pl.ds(c * 128, 128)], swa)
            pltpu.async_copy(
                b3, d_hbm.at[pl.ds(e00 + chunk, chunk), pl.ds(c * 128, 128)],
                swb)

            def issue2(kk, b):
                ahg, hmg, isv, irv, sem, sw = p2bufs[b]
                e0 = g * epg + s * ept + kk * chunk
                pltpu.make_async_copy(
                    ahg, d_hbm.at[pl.ds(0, chunk), pl.ds(0, 128)], sw).wait()
                pltpu.async_copy(src_hbm.at[pl.ds(e0, chunk)], isv, sem)
                pltpu.async_copy(rev_hbm.at[pl.ds(e0, chunk)], irv, sem)
                pltpu.make_async_copy(src_hbm.at[pl.ds(0, chunk)], isv,
                                      sem).wait()
                pltpu.make_async_copy(rev_hbm.at[pl.ds(0, chunk)], irv,
                                      sem).wait()

                aoff = c * ng_all + g * _NGP

                @pl.loop(0, chunk // 16)
                def _(j):
                    sl = pl.ds(j * 16, 16)
                    irv[sl] = irv[sl] * 2 + c
                    isv[sl] = isv[sl] + aoff

                pltpu.async_copy(ah2_hbm.at[isv], ahg, sem)
                pltpu.async_copy(hm2_hbm.at[irv], hmg, sem)

            def drain2(kk, b):
                ahg, hmg, isv, irv, sem, sw = p2bufs[b]
                pltpu.make_async_copy(ah2_hbm.at[isv], ahg, sem).wait()
                pltpu.make_async_copy(hm2_hbm.at[irv], hmg, sem).wait()

                @pl.loop(0, chunk)
                def _(i):
                    for j in range(8):
                        sl = pl.ds(j * 16, 16)
                        ahg[i, sl] = ahg[i, sl] - hmg[i, sl]

                e0 = g * epg + s * ept + kk * chunk
                pltpu.async_copy(
                    ahg, d_hbm.at[pl.ds(e0, chunk), pl.ds(c * 128, 128)], sw)

            issue2(0, 0)
            issue2(1, 1)

            @pl.loop(0, (nchunk + 1) // 2)
            def _(q):
                kk = 2 * q
                drain2(kk, 0)

                @pl.when(kk + 2 < nchunk)
                def _():
                    issue2(kk + 2, 0)

                @pl.when(kk + 1 < nchunk)
                def _():
                    drain2(kk + 1, 1)

                @pl.when(kk + 3 < nchunk)
                def _():
                    issue2(kk + 3, 1)

            pltpu.make_async_copy(
                b1, d_hbm.at[pl.ds(0, chunk), pl.ds(0, 128)], swa).wait()
            pltpu.make_async_copy(
                b3, d_hbm.at[pl.ds(0, chunk), pl.ds(0, 128)], swb).wait()
            plsc.subcore_barrier()

    return k(hm, dst_local, src_local, rev_g, hm2)[0]


def _gatherdiff_sc(ah, hm, src_g, rev_g, e_tot):
    nw = _N_CORES * _N_SUBCORES
    ew = e_tot // nw
    chunk = 80
    nchunk = ew // chunk
    mesh = plsc.VectorSubcoreMesh(core_axis_name="c", subcore_axis_name="s")

    @functools.partial(
        pl.kernel,
        out_type=jax.ShapeDtypeStruct((e_tot, _H), jnp.float32),
        mesh=mesh,
        scratch_types=[
            pltpu.VMEM((chunk, _H), jnp.float32),
            pltpu.VMEM((chunk, _H), jnp.float32),
            pltpu.VMEM((chunk, _H), jnp.float32),
            pltpu.VMEM((chunk, _H), jnp.float32),
            pltpu.VMEM((chunk, _H), jnp.float32),
            pltpu.VMEM((chunk, _H), jnp.float32),
            pltpu.VMEM((chunk,), jnp.int32),
            pltpu.VMEM((chunk,), jnp.int32),
            pltpu.VMEM((chunk,), jnp.int32),
            pltpu.VMEM((chunk,), jnp.int32),
            pltpu.SemaphoreType.DMA,
            pltpu.SemaphoreType.DMA,
            pltpu.SemaphoreType.DMA,
            pltpu.SemaphoreType.DMA,
        ],
    )
    def k(ah_hbm, hm_hbm, src_hbm, rev_hbm, out_hbm,
          g1_a, g1_b, g2_a, g2_b, o_a, o_b,
          is_a, is_b, ir_a, ir_b, sg_a, sg_b, sw_a, sw_b):
        c = lax.axis_index("c")
        s = lax.axis_index("s")
        w = s * _N_CORES + c
        bufs = ((g1_a, g2_a, o_a, is_a, ir_a, sg_a, sw_a),
                (g1_b, g2_b, o_b, is_b, ir_b, sg_b, sw_b))

        pltpu.async_copy(o_a, out_hbm.at[pl.ds(w * ew, chunk)], sw_a)
        pltpu.async_copy(o_b, out_hbm.at[pl.ds(w * ew + chunk, chunk)], sw_b)

        def issue(kk, b):
            g1, g2, o, isv, irv, sg, sw = bufs[b]
            e0 = w * ew + kk * chunk
            pltpu.async_copy(src_hbm.at[pl.ds(e0, chunk)], isv, sg)
            pltpu.async_copy(rev_hbm.at[pl.ds(e0, chunk)], irv, sg)
            pltpu.make_async_copy(src_hbm.at[pl.ds(0, chunk)], isv, sg).wait()
            pltpu.make_async_copy(rev_hbm.at[pl.ds(0, chunk)], irv, sg).wait()
            pltpu.async_copy(ah_hbm.at[isv], g1, sg)
            pltpu.async_copy(hm_hbm.at[irv], g2, sg)

        def drain(kk, b):
            g1, g2, o, isv, irv, sg, sw = bufs[b]
            pltpu.make_async_copy(ah_hbm.at[isv], g1, sg).wait()
            pltpu.make_async_copy(hm_hbm.at[irv], g2, sg).wait()
            pltpu.make_async_copy(o, out_hbm.at[pl.ds(0, chunk)], sw).wait()

            @pl.loop(0, chunk)
            def _(i):
                for j in range(_H // 16):
                    sl = pl.ds(j * 16, 16)
                    o[i, sl] = g1[i, sl] - g2[i, sl]

            e0 = w * ew + kk * chunk
            pltpu.async_copy(o, out_hbm.at[pl.ds(e0, chunk)], sw)

        issue(0, 0)
        issue(1, 1)

        @pl.loop(0, (nchunk + 1) // 2)
        def _(q):
            kk = 2 * q
            drain(kk, 0)

            @pl.when(kk + 2 < nchunk)
            def _():
                issue(kk + 2, 0)

            @pl.when(kk + 1 < nchunk)
            def _():
                drain(kk + 1, 1)

            @pl.when(kk + 3 < nchunk)
            def _():
                issue(kk + 3, 1)

        pltpu.make_async_copy(o_a, out_hbm.at[pl.ds(0, chunk)], sw_a).wait()
        pltpu.make_async_copy(o_b, out_hbm.at[pl.ds(0, chunk)], sw_b).wait()

    return k(ah, hm, src_g, rev_g)


def _out_body(fa_ref, ah_ref, mol_ref, woa_ref, wob_ref, o_ref, sums, cnts,
              *, nblk):
    i = pl.program_id(0)
    hid = jnp.maximum(
        jnp.dot(fa_ref[...], woa_ref[...], preferred_element_type=jnp.float32)
        + jnp.dot(ah_ref[...], wob_ref[...], preferred_element_type=jnp.float32),
        0.0)
    m = sums.shape[0]
    b = hid.shape[0]
    mol = mol_ref[0, 0, :]
    onehot = (lax.broadcasted_iota(jnp.int32, (m, b), 0)
              == mol[None, :]).astype(jnp.float32)

    @pl.when(i == 0)
    def _():
        sums[...] = jnp.zeros_like(sums)
        cnts[...] = jnp.zeros_like(cnts)

    sums[...] += jnp.dot(onehot, hid, preferred_element_type=jnp.float32)
    cnts[...] += jnp.dot(onehot, jnp.ones_like(hid),
                         preferred_element_type=jnp.float32)

    @pl.when(i == nblk - 1)
    def _():
        o_ref[...] = sums[...] / jnp.maximum(cnts[...], 1.0)


def _readout(f_atoms, am, mol3d, w_oa, w_ob, num_graphs, block=1024):
    na = f_atoms.shape[0]
    m = num_graphs * _MG
    nblk = na // block
    return pl.pallas_call(
        functools.partial(_out_body, nblk=nblk),
        grid=(nblk,),
        in_specs=[
            pl.BlockSpec((block, _AF), lambda i: (i, 0)),
            pl.BlockSpec((block, _H), lambda i: (i, 0)),
            pl.BlockSpec((1, 1, block), lambda i: (i, 0, 0)),
            pl.BlockSpec((_AF, _H), lambda i: (0, 0)),
            pl.BlockSpec((_H, _H), lambda i: (0, 0)),
        ],
        out_specs=pl.BlockSpec((m, _H), lambda i: (0, 0)),
        out_shape=jax.ShapeDtypeStruct((m, _H), jnp.float32),
        scratch_shapes=[pltpu.VMEM((m, _H), jnp.float32),
                        pltpu.VMEM((m, _H), jnp.float32)],
    )(f_atoms, am, mol3d, w_oa, w_ob)


def _encode(inp, f_atoms, src_local, dst_local, rev_g, mol3d,
            w_h, w_oa, w_ob, num_graphs):
    d = None
    for t in range(_DEPTH - 1):
        if t == 0:
            hm, hm2 = _mm_h2(inp, w_h, relu_in=True)
        else:
            hm, hm2 = _mm_h2(inp, w_h, d=d)
        d = _segdiff_sc(hm, hm2, dst_local, src_local, rev_g, num_graphs)
    am = _segsum_sc(_msg(inp, d), dst_local, num_graphs)
    return _readout(f_atoms, am, mol3d, w_oa, w_ob, num_graphs)


def _pad_atoms(f_atoms, mol_ids):
    pad = _NGP - _NG
    fa = jnp.concatenate(
        [f_atoms, jnp.zeros((pad, f_atoms.shape[1]), f_atoms.dtype)], axis=0)
    mol = jnp.concatenate([mol_ids, jnp.full((pad,), -1, jnp.int32)])
    return fa, mol


def kernel(f_atoms, f_bonds, edge_index, b2revb, mol_ids,
           f_atoms_next, f_bonds_next, edge_index_next, b2revb_next,
           mol_ids_next, f_atoms_neg, f_bonds_neg, edge_index_neg,
           b2revb_neg, mol_ids_neg, W_i1, W_h1, W_o1, W_i2, W_h2, W_o2):
    fa1, mol1 = _pad_atoms(f_atoms, mol_ids)
    inp1 = _mm(f_bonds, W_i1)
    out1 = _encode(inp1, fa1, edge_index[0], edge_index[1], b2revb,
                   mol1.reshape(_NGP // 1024, 1, 1024),
                   W_h1, W_o1[:_AF], W_o1[_AF:], 1)

    fa_b, mol_b = _pad_atoms(f_atoms_next, mol_ids_next)
    fa_c, mol_c = _pad_atoms(f_atoms_neg, mol_ids_neg)
    fa2 = jnp.concatenate([fa_b, fa_c], axis=0)
    src2 = jnp.concatenate([edge_index_next[0], edge_index_neg[0]])
    dst2 = jnp.concatenate([edge_index_next[1], edge_index_neg[1]])
    rev2 = jnp.concatenate([b2revb_next, b2revb_neg + _EG])
    mol2 = jnp.concatenate(
        [mol_b, jnp.where(mol_c >= 0, mol_c + _MG, mol_c)]) \
        .reshape(2 * _NGP // 1024, 1, 1024)
    inp2 = _mm_i2(f_bonds_next, f_bonds_neg, W_i2)
    out2 = _encode(inp2, fa2, src2, dst2, rev2, mol2,
                   W_h2, W_o2[:_AF], W_o2[_AF:], 2)

    return (out1, out2[:_MG], out2[_MG:])


# speedup vs baseline: 1.1221x; 1.1221x over previous
import functools

import jax
import jax.numpy as jnp
from jax import lax
from jax.experimental import pallas as pl
from jax.experimental.pallas import tpu as pltpu
from jax.experimental.pallas import tpu_sc as plsc

_H = 256
_NG = 10000
_NGP = 10240
_EG = 320000
_MG = 512
_AF = 128
_BF = 144
_DEPTH = 3

_N_SUBCORES = 16
_N_CORES = 2


def _bdot(x, w):
    return jnp.dot(x.astype(jnp.bfloat16), w.astype(jnp.bfloat16),
                   preferred_element_type=jnp.float32)


def _mm_body(x_ref, w_ref, o_ref, *, relu_in):
    x = x_ref[...]
    if relu_in:
        x = jnp.maximum(x, 0.0)
    o_ref[...] = _bdot(x, w_ref[...])


def _mm(x, w, relu_in=False, block=2000):
    e, k = x.shape
    n = w.shape[1]
    return pl.pallas_call(
        functools.partial(_mm_body, relu_in=relu_in),
        grid=(e // block,),
        in_specs=[pl.BlockSpec((block, k), lambda i: (i, 0)),
                  pl.BlockSpec((k, n), lambda i: (0, 0))],
        out_specs=pl.BlockSpec((block, n), lambda i: (i, 0)),
        out_shape=jax.ShapeDtypeStruct((e, n), jnp.float32),
    )(x, w)


def _mm_i2_body(a_ref, b_ref, w_ref, o_ref, *, nblk):
    sel = pl.program_id(0) < nblk
    x = jnp.where(sel, a_ref[...], b_ref[...])
    o_ref[...] = _bdot(x, w_ref[...])


def _mm_i2(xa, xb, w, block=2000):
    e, k = xa.shape
    n = w.shape[1]
    nblk = e // block
    return pl.pallas_call(
        functools.partial(_mm_i2_body, nblk=nblk),
        grid=(2 * nblk,),
        in_specs=[
            pl.BlockSpec((block, k),
                         lambda i: (jnp.where(i < nblk, i, 0), 0)),
            pl.BlockSpec((block, k),
                         lambda i: (jnp.where(i >= nblk, i - nblk, 0), 0)),
            pl.BlockSpec((k, n), lambda i: (0, 0)),
        ],
        out_specs=pl.BlockSpec((block, n), lambda i: (i, 0)),
        out_shape=jax.ShapeDtypeStruct((2 * e, n), jnp.float32),
    )(xa, xb, w)


def _mm_hd_body(inp_ref, d_ref, w_ref, o_ref):
    x = jnp.maximum(inp_ref[...] + d_ref[...], 0.0)
    o_ref[...] = _bdot(x, w_ref[...])


def _mm_hd(inp, d, w, block=2000):
    e, k = inp.shape
    n = w.shape[1]
    return pl.pallas_call(
        _mm_hd_body,
        grid=(e // block,),
        in_specs=[pl.BlockSpec((block, k), lambda i: (i, 0)),
                  pl.BlockSpec((block, k), lambda i: (i, 0)),
                  pl.BlockSpec((k, n), lambda i: (0, 0))],
        out_specs=pl.BlockSpec((block, n), lambda i: (i, 0)),
        out_shape=jax.ShapeDtypeStruct((e, n), jnp.float32),
    )(inp, d, w)


def _msg_body(inp_ref, d_ref, o_ref):
    o_ref[...] = jnp.maximum(inp_ref[...] + d_ref[...], 0.0)


def _msg(inp, d, block=2000):
    e, k = inp.shape
    return pl.pallas_call(
        _msg_body,
        grid=(e // block,),
        in_specs=[pl.BlockSpec((block, k), lambda i: (i, 0)),
                  pl.BlockSpec((block, k), lambda i: (i, 0))],
        out_specs=pl.BlockSpec((block, k), lambda i: (i, 0)),
        out_shape=jax.ShapeDtypeStruct((e, k), jnp.float32),
    )(inp, d)


def _zero_stripe(buf, accum, s, npt, zrows):
    @pl.loop(0, zrows)
    def _(i):
        for j in range(8):
            buf[i, pl.ds(j * 16, 16)] = jnp.zeros((16,), jnp.float32)

    for z in range(npt // zrows):
        pltpu.sync_copy(buf.at[pl.ds(0, zrows)],
                        accum.at[pl.ds(s * npt + z * zrows, zrows)])


def _segsum_sc(hm, dst_local, num_graphs):
    e_tot = hm.shape[0]
    epg = e_tot // num_graphs
    ept = epg // _N_SUBCORES
    chunk = 160
    nchunk = ept // chunk
    npt = _NGP // _N_SUBCORES
    mesh = plsc.VectorSubcoreMesh(core_axis_name="c", subcore_axis_name="s")

    @functools.partial(
        pl.kernel,
        out_type=jax.ShapeDtypeStruct((num_graphs * _NGP, _H), jnp.float32),
        mesh=mesh,
        scratch_types=[
            pltpu.VMEM_SHARED((_NGP, 128), jnp.float32),
            pltpu.VMEM((chunk, 128), jnp.float32),
            pltpu.VMEM((chunk, 128), jnp.float32),
            pltpu.VMEM((chunk,), jnp.int32),
            pltpu.VMEM((chunk,), jnp.int32),
            pltpu.SemaphoreType.DMA,
            pltpu.SemaphoreType.DMA,
        ],
    )
    def k(hm_hbm, dst_hbm, out_hbm, accum,
          rows_a, rows_b, idx_a, idx_b, sem_a, sem_b):
        c = lax.axis_index("c")
        s = lax.axis_index("s")
        bufs = ((rows_a, idx_a, sem_a), (rows_b, idx_b, sem_b))

        for g in range(num_graphs):
            _zero_stripe(rows_a, accum, s, npt, chunk)
            plsc.subcore_barrier()

            def issue(kk, b):
                rows, idx, sem = bufs[b]
                e0 = g * epg + s * ept + kk * chunk
                pltpu.async_copy(dst_hbm.at[pl.ds(e0, chunk)], idx, sem)
                pltpu.async_copy(
                    hm_hbm.at[pl.ds(e0, chunk), pl.ds(c * 128, 128)],
                    rows, sem)

            def drain(b):
                rows, idx, sem = bufs[b]
                pltpu.make_async_copy(dst_hbm.at[pl.ds(0, chunk)], idx,
                                      sem).wait()
                pltpu.make_async_copy(
                    hm_hbm.at[pl.ds(0, chunk), pl.ds(0, 128)], rows,
                    sem).wait()
                pltpu.sync_copy(rows, accum.at[idx], add=True)

            issue(0, 0)
            issue(1, 1)

            @pl.loop(0, (nchunk + 1) // 2)
            def _(q):
                kk = 2 * q
                drain(0)

                @pl.when(kk + 2 < nchunk)
                def _():
                    issue(kk + 2, 0)

                @pl.when(kk + 1 < nchunk)
                def _():
                    drain(1)

                @pl.when(kk + 3 < nchunk)
                def _():
                    issue(kk + 3, 1)

            plsc.subcore_barrier()
            pltpu.sync_copy(
                accum.at[pl.ds(s * npt, npt)],
                out_hbm.at[pl.ds(g * _NGP + s * npt, npt), pl.ds(c * 128, 128)])
            plsc.subcore_barrier()

    return k(hm, dst_local)


def _gatherdiff_sc(ah, hm, src_g, rev_g, e_tot):
    nw = _N_CORES * _N_SUBCORES
    ew = e_tot // nw
    chunk = 80
    nchunk = ew // chunk
    mesh = plsc.VectorSubcoreMesh(core_axis_name="c", subcore_axis_name="s")

    @functools.partial(
        pl.kernel,
        out_type=jax.ShapeDtypeStruct((e_tot, _H), jnp.float32),
        mesh=mesh,
        scratch_types=[
            pltpu.VMEM((chunk, _H), jnp.float32),
            pltpu.VMEM((chunk, _H), jnp.float32),
            pltpu.VMEM((chunk, _H), jnp.float32),
            pltpu.VMEM((chunk, _H), jnp.float32),
            pltpu.VMEM((chunk, _H), jnp.float32),
            pltpu.VMEM((chunk, _H), jnp.float32),
            pltpu.VMEM((chunk,), jnp.int32),
            pltpu.VMEM((chunk,), jnp.int32),
            pltpu.VMEM((chunk,), jnp.int32),
            pltpu.VMEM((chunk,), jnp.int32),
            pltpu.SemaphoreType.DMA,
            pltpu.SemaphoreType.DMA,
            pltpu.SemaphoreType.DMA,
            pltpu.SemaphoreType.DMA,
        ],
    )
    def k(ah_hbm, hm_hbm, src_hbm, rev_hbm, out_hbm,
          g1_a, g1_b, g2_a, g2_b, o_a, o_b,
          is_a, is_b, ir_a, ir_b, sg_a, sg_b, sw_a, sw_b):
        c = lax.axis_index("c")
        s = lax.axis_index("s")
        w = s * _N_CORES + c
        bufs = ((g1_a, g2_a, o_a, is_a, ir_a, sg_a, sw_a),
                (g1_b, g2_b, o_b, is_b, ir_b, sg_b, sw_b))

        pltpu.async_copy(o_a, out_hbm.at[pl.ds(w * ew, chunk)], sw_a)
        pltpu.async_copy(o_b, out_hbm.at[pl.ds(w * ew + chunk, chunk)], sw_b)

        def issue(kk, b):
            g1, g2, o, isv, irv, sg, sw = bufs[b]
            e0 = w * ew + kk * chunk
            pltpu.async_copy(src_hbm.at[pl.ds(e0, chunk)], isv, sg)
            pltpu.async_copy(rev_hbm.at[pl.ds(e0, chunk)], irv, sg)
            pltpu.make_async_copy(src_hbm.at[pl.ds(0, chunk)], isv, sg).wait()
            pltpu.make_async_copy(rev_hbm.at[pl.ds(0, chunk)], irv, sg).wait()
            pltpu.async_copy(ah_hbm.at[isv], g1, sg)
            pltpu.async_copy(hm_hbm.at[irv], g2, sg)

        def drain(kk, b):
            g1, g2, o, isv, irv, sg, sw = bufs[b]
            pltpu.make_async_copy(ah_hbm.at[isv], g1, sg).wait()
            pltpu.make_async_copy(hm_hbm.at[irv], g2, sg).wait()
            pltpu.make_async_copy(o, out_hbm.at[pl.ds(0, chunk)], sw).wait()

            @pl.loop(0, chunk)
            def _(i):
                for j in range(_H // 16):
                    sl = pl.ds(j * 16, 16)
                    o[i, sl] = g1[i, sl] - g2[i, sl]

            e0 = w * ew + kk * chunk
            pltpu.async_copy(o, out_hbm.at[pl.ds(e0, chunk)], sw)

        issue(0, 0)
        issue(1, 1)

        @pl.loop(0, (nchunk + 1) // 2)
        def _(q):
            kk = 2 * q
            drain(kk, 0)

            @pl.when(kk + 2 < nchunk)
            def _():
                issue(kk + 2, 0)

            @pl.when(kk + 1 < nchunk)
            def _():
                drain(kk + 1, 1)

            @pl.when(kk + 3 < nchunk)
            def _():
                issue(kk + 3, 1)

        pltpu.make_async_copy(o_a, out_hbm.at[pl.ds(0, chunk)], sw_a).wait()
        pltpu.make_async_copy(o_b, out_hbm.at[pl.ds(0, chunk)], sw_b).wait()

    return k(ah, hm, src_g, rev_g)


def _out_body(fa_ref, ah_ref, mol_ref, woa_ref, wob_ref, o_ref, sums, cnts,
              *, nblk):
    i = pl.program_id(0)
    hid = jnp.maximum(
        jnp.dot(fa_ref[...], woa_ref[...], preferred_element_type=jnp.float32)
        + jnp.dot(ah_ref[...], wob_ref[...], preferred_element_type=jnp.float32),
        0.0)
    m = sums.shape[0]
    b = hid.shape[0]
    mol = mol_ref[0, 0, :]
    onehot = (lax.broadcasted_iota(jnp.int32, (m, b), 0)
              == mol[None, :]).astype(jnp.float32)

    @pl.when(i == 0)
    def _():
        sums[...] = jnp.zeros_like(sums)
        cnts[...] = jnp.zeros_like(cnts)

    sums[...] += jnp.dot(onehot, hid, preferred_element_type=jnp.float32)
    cnts[...] += jnp.dot(onehot, jnp.ones_like(hid),
                         preferred_element_type=jnp.float32)

    @pl.when(i == nblk - 1)
    def _():
        o_ref[...] = sums[...] / jnp.maximum(cnts[...], 1.0)


def _readout(f_atoms, am, mol3d, w_oa, w_ob, num_graphs, block=1024):
    na = f_atoms.shape[0]
    m = num_graphs * _MG
    nblk = na // block
    return pl.pallas_call(
        functools.partial(_out_body, nblk=nblk),
        grid=(nblk,),
        in_specs=[
            pl.BlockSpec((block, _AF), lambda i: (i, 0)),
            pl.BlockSpec((block, _H), lambda i: (i, 0)),
            pl.BlockSpec((1, 1, block), lambda i: (i, 0, 0)),
            pl.BlockSpec((_AF, _H), lambda i: (0, 0)),
            pl.BlockSpec((_H, _H), lambda i: (0, 0)),
        ],
        out_specs=pl.BlockSpec((m, _H), lambda i: (0, 0)),
        out_shape=jax.ShapeDtypeStruct((m, _H), jnp.float32),
        scratch_shapes=[pltpu.VMEM((m, _H), jnp.float32),
                        pltpu.VMEM((m, _H), jnp.float32)],
    )(f_atoms, am, mol3d, w_oa, w_ob)


def _encode(inp, f_atoms, src_g, dst_local, rev_g, mol3d,
            w_h, w_oa, w_ob, num_graphs):
    e_tot = inp.shape[0]
    hm = _mm(inp, w_h, relu_in=True)
    d = None
    for t in range(_DEPTH - 1):
        if t > 0:
            hm = _mm_hd(inp, d, w_h)
        ah = _segsum_sc(hm, dst_local, num_graphs)
        d = _gatherdiff_sc(ah, hm, src_g, rev_g, e_tot)
    am = _segsum_sc(_msg(inp, d), dst_local, num_graphs)
    return _readout(f_atoms, am, mol3d, w_oa, w_ob, num_graphs)


def _pad_atoms(f_atoms, mol_ids):
    pad = _NGP - _NG
    fa = jnp.concatenate(
        [f_atoms, jnp.zeros((pad, f_atoms.shape[1]), f_atoms.dtype)], axis=0)
    mol = jnp.concatenate([mol_ids, jnp.full((pad,), -1, jnp.int32)])
    return fa, mol


def kernel(f_atoms, f_bonds, edge_index, b2revb, mol_ids,
           f_atoms_next, f_bonds_next, edge_index_next, b2revb_next,
           mol_ids_next, f_atoms_neg, f_bonds_neg, edge_index_neg,
           b2revb_neg, mol_ids_neg, W_i1, W_h1, W_o1, W_i2, W_h2, W_o2):
    fa1, mol1 = _pad_atoms(f_atoms, mol_ids)
    inp1 = _mm(f_bonds, W_i1)
    out1 = _encode(inp1, fa1, edge_index[0], edge_index[1], b2revb,
                   mol1.reshape(_NGP // 1024, 1, 1024),
                   W_h1, W_o1[:_AF], W_o1[_AF:], 1)

    fa_b, mol_b = _pad_atoms(f_atoms_next, mol_ids_next)
    fa_c, mol_c = _pad_atoms(f_atoms_neg, mol_ids_neg)
    fa2 = jnp.concatenate([fa_b, fa_c], axis=0)
    src2 = jnp.concatenate([edge_index_next[0], edge_index_neg[0] + _NGP])
    dst2 = jnp.concatenate([edge_index_next[1], edge_index_neg[1]])
    rev2 = jnp.concatenate([b2revb_next, b2revb_neg + _EG])
    mol2 = jnp.concatenate(
        [mol_b, jnp.where(mol_c >= 0, mol_c + _MG, mol_c)]) \
        .reshape(2 * _NGP // 1024, 1, 1024)
    inp2 = _mm_i2(f_bonds_next, f_bonds_neg, W_i2)
    out2 = _encode(inp2, fa2, src2, dst2, rev2, mol2,
                   W_h2, W_o2[:_AF], W_o2[_AF:], 2)

    return (out1, out2[:_MG], out2[_MG:])
